# Initial kernel scaffold; baseline (speedup 1.0000x reference)
import jax
import jax.numpy as jnp
from jax.experimental import pallas as pl


def kernel(colors, pixel_coords_cameras, background_mask):
    raise NotImplementedError("write your pallas kernel here")


# speedup vs baseline: 118.4050x; 118.4050x over previous
import jax
import jax.numpy as jnp
import numpy as np
from jax.experimental import pallas as pl
from jax.experimental.pallas import tpu as pltpu

_N, _H, _W, _K = 2, 224, 224, 4
_SIGMA = 0.5
_FOV = 60.0
_ZNEAR = 1.0
_ZFAR = 100.0
_BG_COLOR = (1.0, 1.0, 1.0)
_OFFSETS = [(-1, -1), (-1, 0), (-1, 1), (0, -1), (0, 0), (0, 1), (1, -1), (1, 0), (1, 1)]
_BIG_DEPTH = 1.0e7
_EPS = 1e-8

_TH = 56
_NT = _H // _TH


def _splat_kernel(inp_hbm, out_ref, scratch, sem):
    n = pl.program_id(0)
    t = pl.program_id(1)
    cp = pltpu.make_async_copy(
        inp_hbm.at[n, :, pl.ds(t * _TH, 64), :], scratch, sem)
    cp.start()
    cp.wait()

    H, W = _TH, _W
    s = 1.0 / np.tan(np.radians(_FOV) / 2.0)
    zc1 = _ZFAR / (_ZFAR - _ZNEAR)
    inv2s2 = 1.0 / (2.0 * _SIGMA * _SIGMA)

    def crop(p, dy=0, dx=0):
        return jax.lax.slice(p, (1 - dy, 1 - dx), (1 - dy + H, 1 - dx + W))

    zn = []
    exs, eys = [], []
    cols = []
    for k in range(_K):
        x = scratch[k]
        y = scratch[4 + k]
        z = scratch[8 + k]
        bg = scratch[28 + k] > 0.5
        zmax = jnp.maximum(z, 1e-6)
        xs = (_W - 1) / 2.0 * (1.0 - x * s / zmax) + 0.5
        ys = (_H - 1) / 2.0 * (1.0 - y * s / zmax) + 0.5
        zk = zc1 * (1.0 - _ZNEAR / zmax)
        xs = jnp.where(bg, 0.0, xs)
        ys = jnp.where(bg, 0.0, ys)
        zk = jnp.where(bg, _BIG_DEPTH, zk)
        zn.append(zk)
        fx = xs - jnp.floor(xs) - 0.5
        fy = ys - jnp.floor(ys) - 0.5
        exs.append([jnp.exp(-(fx - d) * (fx - d) * inv2s2) for d in (-1.0, 0.0, 1.0)])
        eys.append([jnp.exp(-(fy - d) * (fy - d) * inv2s2) for d in (-1.0, 0.0, 1.0)])
        ck = [scratch[12 + 4 * k + c] for c in range(4)]
        cols.append([jnp.where(bg, 0.0, c) for c in ck])

    qtop = crop(zn[0])

    zero = jnp.zeros((H, W), dtype=jnp.float32)
    acc = [[zero for _ in range(3)] for _ in range(5)]

    for (dy, dx) in _OFFSETS:
        dyi, dxi = dy + 1, dx + 1
        diffs = [jnp.abs(crop(zn[k], dy, dx) - qtop) for k in range(_K)]
        kstar = zero
        best = diffs[0]
        for k in range(1, _K):
            better = diffs[k] < best
            kstar = jnp.where(better, float(k), kstar)
            best = jnp.where(better, diffs[k], best)
        for k in range(_K):
            wk = cols[k][3] * exs[k][dxi] * eys[k][dyi]
            vals = [crop(wk * cols[k][c], dy, dx) for c in range(3)]
            vals.append(crop(wk * cols[k][3], dy, dx))
            vals.append(crop(wk, dy, dx))
            kf = float(k)
            surf = (kstar == kf).astype(jnp.float32)
            bgm = (kstar < kf).astype(jnp.float32)
            fgm = 1.0 - surf - bgm
            for c in range(5):
                acc[c][0] = acc[c][0] + vals[c] * bgm
                acc[c][1] = acc[c][1] + vals[c] * surf
                acc[c][2] = acc[c][2] + vals[c] * fgm

    outp = [jnp.full((H, W), c, dtype=jnp.float32) for c in (*_BG_COLOR, 1.0)]
    for bucket in range(3):
        winv = 1.0 / jnp.maximum(acc[4][bucket], _EPS)
        a = acc[3][bucket] * winv
        one_minus_a = 1.0 - a
        for c in range(4):
            outp[c] = acc[c][bucket] * winv + one_minus_a * outp[c]
    for c in range(4):
        out_ref[0, c] = outp[c]


@jax.jit
def kernel(colors, pixel_coords_cameras, background_mask):
    pcs = jnp.transpose(pixel_coords_cameras, (0, 4, 3, 1, 2))
    pcs = pcs.reshape(_N, 12, _H, _W)
    col = jnp.transpose(colors, (0, 3, 4, 1, 2)).reshape(_N, 16, _H, _W)
    bg = jnp.transpose(background_mask, (0, 3, 1, 2)).astype(jnp.float32)
    pad = lambda a, v: jnp.pad(a, ((0, 0), (0, 0), (1, 7), (1, 1)), constant_values=v)
    inp = jnp.concatenate([pad(pcs, 0.0), pad(col, 0.0), pad(bg, 1.0)], axis=1)

    out = pl.pallas_call(
        _splat_kernel,
        grid=(_N, _NT),
        in_specs=[pl.BlockSpec(memory_space=pl.ANY)],
        out_specs=pl.BlockSpec((1, 4, _TH, _W), lambda n, t: (n, 0, t, 0)),
        out_shape=jax.ShapeDtypeStruct((_N, 4, _H, _W), jnp.float32),
        scratch_shapes=[
            pltpu.VMEM((32, 64, _W + 2), jnp.float32),
            pltpu.SemaphoreType.DMA,
        ],
    )(inp)
    return jnp.transpose(out, (0, 2, 3, 1))


# speedup vs baseline: 130.4272x; 1.1015x over previous
import jax
import jax.numpy as jnp
import numpy as np
from jax.experimental import pallas as pl
from jax.experimental.pallas import tpu as pltpu

_N, _H, _W, _K = 2, 224, 224, 4
_SIGMA = 0.5
_FOV = 60.0
_ZNEAR = 1.0
_ZFAR = 100.0
_BG_COLOR = (1.0, 1.0, 1.0)
_OFFSETS = [(-1, -1), (-1, 0), (-1, 1), (0, -1), (0, 0), (0, 1), (1, -1), (1, 0), (1, 1)]
_BIG_DEPTH = 1.0e7
_EPS = 1e-8

_TH = 56
_NT = _H // _TH


def _splat_kernel(inp_hbm, out_ref, scratch, sem):
    n = pl.program_id(0)
    t = pl.program_id(1)
    step = n * _NT + t
    par = jax.lax.rem(step, 2)

    def copy_for(nn, tt, p):
        return pltpu.make_async_copy(
            inp_hbm.at[nn, :, pl.ds(tt * _TH, 64), :], scratch.at[p], sem.at[p])

    @pl.when(step == 0)
    def _():
        copy_for(n, t, par).start()

    nxt = step + 1
    @pl.when(nxt < _N * _NT)
    def _():
        copy_for(nxt // _NT, jax.lax.rem(nxt, _NT), jax.lax.rem(nxt, 2)).start()

    copy_for(n, t, par).wait()
    buf = scratch.at[par]

    H, W = _TH, _W
    s = 1.0 / np.tan(np.radians(_FOV) / 2.0)
    zc1 = _ZFAR / (_ZFAR - _ZNEAR)
    inv2s2 = 1.0 / (2.0 * _SIGMA * _SIGMA)

    def crop(p, dy=0, dx=0):
        return jax.lax.slice(p, (1 - dy, 1 - dx), (1 - dy + H, 1 - dx + W))

    zn = []
    exs, eys = [], []
    cols = []
    for k in range(_K):
        x = buf[k]
        y = buf[4 + k]
        z = buf[8 + k]
        bg = buf[28 + k] > 0.5
        zmax = jnp.maximum(z, 1e-6)
        xs = (_W - 1) / 2.0 * (1.0 - x * s / zmax) + 0.5
        ys = (_H - 1) / 2.0 * (1.0 - y * s / zmax) + 0.5
        zk = zc1 * (1.0 - _ZNEAR / zmax)
        xs = jnp.where(bg, 0.0, xs)
        ys = jnp.where(bg, 0.0, ys)
        zk = jnp.where(bg, _BIG_DEPTH, zk)
        zn.append(zk)
        fx = xs - jnp.floor(xs) - 0.5
        fy = ys - jnp.floor(ys) - 0.5
        exs.append([jnp.exp(-(fx - d) * (fx - d) * inv2s2) for d in (-1.0, 0.0, 1.0)])
        eys.append([jnp.exp(-(fy - d) * (fy - d) * inv2s2) for d in (-1.0, 0.0, 1.0)])
        ck = [buf[12 + 4 * k + c] for c in range(4)]
        cols.append([jnp.where(bg, 0.0, c) for c in ck])

    qtop = crop(zn[0])

    zero = jnp.zeros((H, W), dtype=jnp.float32)
    acc = [[zero for _ in range(3)] for _ in range(5)]

    for (dy, dx) in _OFFSETS:
        dyi, dxi = dy + 1, dx + 1
        diffs = [jnp.abs(crop(zn[k], dy, dx) - qtop) for k in range(_K)]
        kstar = zero
        best = diffs[0]
        for k in range(1, _K):
            better = diffs[k] < best
            kstar = jnp.where(better, float(k), kstar)
            best = jnp.where(better, diffs[k], best)
        for k in range(_K):
            wk = cols[k][3] * exs[k][dxi] * eys[k][dyi]
            vals = [crop(wk * cols[k][c], dy, dx) for c in range(3)]
            vals.append(crop(wk * cols[k][3], dy, dx))
            vals.append(crop(wk, dy, dx))
            kf = float(k)
            surf = (kstar == kf).astype(jnp.float32)
            bgm = (kstar < kf).astype(jnp.float32)
            fgm = 1.0 - surf - bgm
            for c in range(5):
                acc[c][0] = acc[c][0] + vals[c] * bgm
                acc[c][1] = acc[c][1] + vals[c] * surf
                acc[c][2] = acc[c][2] + vals[c] * fgm

    outp = [jnp.full((H, W), c, dtype=jnp.float32) for c in (*_BG_COLOR, 1.0)]
    for bucket in range(3):
        winv = 1.0 / jnp.maximum(acc[4][bucket], _EPS)
        a = acc[3][bucket] * winv
        one_minus_a = 1.0 - a
        for c in range(4):
            outp[c] = acc[c][bucket] * winv + one_minus_a * outp[c]
    for c in range(4):
        out_ref[0, c] = outp[c]


@jax.jit
def kernel(colors, pixel_coords_cameras, background_mask):
    pcs = jnp.transpose(pixel_coords_cameras, (0, 4, 3, 1, 2))
    pcs = pcs.reshape(_N, 12, _H, _W)
    col = jnp.transpose(colors, (0, 3, 4, 1, 2)).reshape(_N, 16, _H, _W)
    bg = jnp.transpose(background_mask, (0, 3, 1, 2)).astype(jnp.float32)
    pad = lambda a, v: jnp.pad(a, ((0, 0), (0, 0), (1, 7), (1, 1)), constant_values=v)
    inp = jnp.concatenate([pad(pcs, 0.0), pad(col, 0.0), pad(bg, 1.0)], axis=1)

    out = pl.pallas_call(
        _splat_kernel,
        grid=(_N, _NT),
        in_specs=[pl.BlockSpec(memory_space=pl.ANY)],
        out_specs=pl.BlockSpec((1, 4, _TH, _W), lambda n, t: (n, 0, t, 0)),
        out_shape=jax.ShapeDtypeStruct((_N, 4, _H, _W), jnp.float32),
        scratch_shapes=[
            pltpu.VMEM((2, 32, 64, _W + 2), jnp.float32),
            pltpu.SemaphoreType.DMA((2,)),
        ],
    )(inp)
    return jnp.transpose(out, (0, 2, 3, 1))
